# baseline (device time: 184006 ns/iter reference)
import numpy as np

import jax
import jax.numpy as jnp
from jax import lax
from jax.experimental import pallas as pl
from jax.experimental.pallas import tpu as pltpu

N_DEV = 4
SQ = 1024
SKV = 1024
D_MODEL = 1024
H_PER = 8
DH = 128
BLK = 64
SCALE = 0.08838834764831843

_PBLOCKS = [0, 3, 6, 9, 12, 15, 1, 4, 7, 10, 13, 2, 5, 8, 11, 14]
_PERM = np.concatenate([np.arange(b * BLK, (b + 1) * BLK) for b in _PBLOCKS])
_INV_PERM = np.argsort(_PERM)
R0 = 6 * BLK
R1 = 11 * BLK


def _body(x_ref, wq_ref, wo_ref, k_ref, v_ref, out_ref,
          commq, commo, dbias_ref, sendq, recvq, sendo, recvo):
    my = lax.axis_index("i")
    right = lax.rem(my + 1, N_DEV)
    left = lax.rem(my + N_DEV - 1, N_DEV)
    diag = lax.rem(my + 2, N_DEV)

    barrier_sem = pltpu.get_barrier_semaphore()
    for nbr in (left, right, diag):
        pl.semaphore_signal(
            barrier_sem, inc=1,
            device_id=(nbr,), device_id_type=pl.DeviceIdType.MESH,
        )
    pl.semaphore_wait(barrier_sem, 3)

    rb = lax.broadcasted_iota(jnp.int32, (R1 - R0, R1 - R0), 0) // BLK
    cb = lax.broadcasted_iota(jnp.int32, (R1 - R0, R1 - R0), 1) // BLK
    dbias_ref[...] = jnp.where(rb == cb, jnp.float32(0.0), jnp.float32(-1e9))

    def mk(src, comm, slot, ssems, rsems, dev):
        return pltpu.make_async_remote_copy(
            src_ref=src, dst_ref=comm.at[slot],
            send_sem=ssems.at[slot], recv_sem=rsems.at[slot],
            device_id=(dev,), device_id_type=pl.DeviceIdType.MESH,
        )

    rq = [mk(wq_ref, commq, 0, sendq, recvq, right),
          mk(wq_ref, commq, 1, sendq, recvq, left),
          mk(wq_ref, commq, 2, sendq, recvq, diag)]
    ro = [mk(wo_ref, commo, 0, sendo, recvo, right),
          mk(wo_ref, commo, 1, sendo, recvo, left),
          mk(wo_ref, commo, 2, sendo, recvo, diag)]

    for i in (0, 1):
        rq[i].start()
        ro[i].start()

    bf = jnp.bfloat16

    def piece(q, k_main, v_main, kb0, vb0, k_diag, v_diag):
        sa = lax.dot_general(q, kb0, (((1,), (1,)), ((), ())),
                             preferred_element_type=jnp.float32)
        sb = lax.dot_general(q, k_main, (((1,), (1,)), ((), ())),
                             preferred_element_type=jnp.float32)
        sd = lax.dot_general(q, k_diag, (((1,), (1,)), ((), ())),
                             preferred_element_type=jnp.float32) + dbias_ref[...]
        wa, wb_, wd = jnp.exp(sa), jnp.exp(sb), jnp.exp(sd)
        d = (jnp.sum(wa, axis=1, keepdims=True)
             + jnp.sum(wb_, axis=1, keepdims=True)
             + jnp.sum(wd, axis=1, keepdims=True))
        c = (jnp.dot(wa.astype(bf), vb0, preferred_element_type=jnp.float32)
             + jnp.dot(wb_.astype(bf), v_main, preferred_element_type=jnp.float32)
             + jnp.dot(wd.astype(bf), v_diag, preferred_element_type=jnp.float32))
        return (c / d).astype(bf)

    def compute_group(g, wq_at, wo_at):
        def head_body(h, carry):
            gh = g * H_PER + h
            qh = jnp.dot(x_ref[...], wq_at(h),
                         preferred_element_type=jnp.float32
                         ).astype(bf)
            kh = k_ref[gh]
            vh = v_ref[gh]
            s0 = lax.dot_general(qh[0:R0], kh[0:R0],
                                 (((1,), (1,)), ((), ())),
                                 preferred_element_type=jnp.float32)
            w0 = jnp.exp(s0)
            d0 = jnp.sum(w0, axis=1, keepdims=True)
            c0 = jnp.dot(w0.astype(bf), vh[0:R0],
                         preferred_element_type=jnp.float32)
            ctx0 = (c0 / d0).astype(bf)
            ctx1 = piece(qh[R0:R1], kh[R1:SKV], vh[R1:SKV],
                         kh[0:BLK], vh[0:BLK], kh[R0:R1], vh[R0:R1])
            ctx2 = piece(qh[R1:SKV], kh[R0:R1], vh[R0:R1],
                         kh[0:BLK], vh[0:BLK], kh[R1:SKV], vh[R1:SKV])
            ctx = jnp.concatenate([ctx0, ctx1, ctx2], axis=0)
            out_ref[...] = out_ref[...] + jnp.dot(
                ctx, wo_at(h), preferred_element_type=jnp.float32)
            return carry
        lax.fori_loop(0, H_PER, head_body, 0)

    out_ref[...] = jnp.zeros((SQ, D_MODEL), jnp.float32)
    compute_group(my, lambda h: wq_ref[h], lambda h: wo_ref[h])

    for i in (0, 1):
        rq[i].wait_send()
        ro[i].wait_send()
    rq[2].start()
    ro[2].start()

    for slot, off in ((0, N_DEV - 1), (1, 1), (2, 2)):
        rq[slot].wait_recv()
        ro[slot].wait_recv()
        g = lax.rem(my + off, N_DEV)
        compute_group(g,
                      lambda h: commq[slot, h],
                      lambda h: commo[slot, h])

    rq[2].wait_send()
    ro[2].wait_send()


def kernel(x, Wq, K_ext, V_ext, Wo):
    my = lax.axis_index("i")
    bf = jnp.bfloat16
    perm = jnp.asarray(_PERM)
    xb = x[0][perm].astype(bf)
    wq3 = jnp.swapaxes(
        (Wq * SCALE).reshape(D_MODEL, H_PER, DH), 0, 1).astype(bf)
    wo3 = Wo.reshape(H_PER, DH, D_MODEL).astype(bf)
    kb = jnp.swapaxes(
        lax.dynamic_index_in_dim(K_ext, my, 0, keepdims=False),
        0, 1)[:, perm, :].astype(bf)
    vb = jnp.swapaxes(
        lax.dynamic_index_in_dim(V_ext, my, 0, keepdims=False),
        0, 1)[:, perm, :].astype(bf)

    out = pl.pallas_call(
        _body,
        out_shape=jax.ShapeDtypeStruct((SQ, D_MODEL), jnp.float32),
        in_specs=[
            pl.BlockSpec(memory_space=pltpu.VMEM),
            pl.BlockSpec(memory_space=pltpu.VMEM),
            pl.BlockSpec(memory_space=pltpu.VMEM),
            pl.BlockSpec(memory_space=pltpu.VMEM),
            pl.BlockSpec(memory_space=pltpu.VMEM),
        ],
        out_specs=pl.BlockSpec(memory_space=pltpu.VMEM),
        scratch_shapes=[
            pltpu.VMEM((N_DEV - 1, H_PER, D_MODEL, DH), bf),
            pltpu.VMEM((N_DEV - 1, H_PER, DH, D_MODEL), bf),
            pltpu.VMEM((R1 - R0, R1 - R0), jnp.float32),
            pltpu.SemaphoreType.DMA((N_DEV - 1,)),
            pltpu.SemaphoreType.DMA((N_DEV - 1,)),
            pltpu.SemaphoreType.DMA((N_DEV - 1,)),
            pltpu.SemaphoreType.DMA((N_DEV - 1,)),
        ],
        compiler_params=pltpu.CompilerParams(collective_id=0),
    )(xb, wq3, wo3, kb, vb)
    return out[jnp.asarray(_INV_PERM)][None]


# device time: 172874 ns/iter; 1.0644x vs baseline; 1.0644x over previous
import numpy as np

import jax
import jax.numpy as jnp
from jax import lax
from jax.experimental import pallas as pl
from jax.experimental.pallas import tpu as pltpu

N_DEV = 4
SQ = 1024
SKV = 1024
D_MODEL = 1024
H_PER = 8
DH = 128
BLK = 64
SCALE = 0.08838834764831843

_B0 = [0, 3, 6, 9, 12, 15]
_B1 = [1, 4, 7, 10, 13]
_B2 = [2, 5, 8, 11, 14]
_PBLOCKS = _B0 + _B1 + _B2
_INV_ORDER = [_PBLOCKS.index(b) for b in range(16)]
R0 = 6 * BLK
R1 = 11 * BLK


def _body(x_ref, wq_ref, wo_ref, k_ref, v_ref, out_ref,
          commq, commo, dbias_ref, sendq, recvq, sendo, recvo):
    my = lax.axis_index("i")
    right = lax.rem(my + 1, N_DEV)
    left = lax.rem(my + N_DEV - 1, N_DEV)
    diag = lax.rem(my + 2, N_DEV)

    barrier_sem = pltpu.get_barrier_semaphore()
    for nbr in (left, right, diag):
        pl.semaphore_signal(
            barrier_sem, inc=1,
            device_id=(nbr,), device_id_type=pl.DeviceIdType.MESH,
        )
    pl.semaphore_wait(barrier_sem, 3)

    rb = lax.broadcasted_iota(jnp.int32, (R1 - R0, R1 - R0), 0) // BLK
    cb = lax.broadcasted_iota(jnp.int32, (R1 - R0, R1 - R0), 1) // BLK
    dbias_ref[...] = jnp.where(rb == cb, jnp.float32(0.0), jnp.float32(-1e9))

    def mk(src, comm, slot, ssems, rsems, dev):
        return pltpu.make_async_remote_copy(
            src_ref=src, dst_ref=comm.at[slot],
            send_sem=ssems.at[slot], recv_sem=rsems.at[slot],
            device_id=(dev,), device_id_type=pl.DeviceIdType.MESH,
        )

    rq = [mk(wq_ref, commq, 0, sendq, recvq, right),
          mk(wq_ref, commq, 1, sendq, recvq, left),
          mk(wq_ref, commq, 2, sendq, recvq, diag)]
    ro = [mk(wo_ref, commo, 0, sendo, recvo, right),
          mk(wo_ref, commo, 1, sendo, recvo, left),
          mk(wo_ref, commo, 2, sendo, recvo, diag)]

    for i in (0, 1):
        rq[i].start()
        ro[i].start()

    bf = jnp.bfloat16

    def piece(q, k_main, v_main, kb0, vb0, k_diag, v_diag):
        sa = lax.dot_general(q, kb0, (((1,), (1,)), ((), ())),
                             preferred_element_type=jnp.float32)
        sb = lax.dot_general(q, k_main, (((1,), (1,)), ((), ())),
                             preferred_element_type=jnp.float32)
        sd = lax.dot_general(q, k_diag, (((1,), (1,)), ((), ())),
                             preferred_element_type=jnp.float32) + dbias_ref[...]
        wa, wb_, wd = jnp.exp(sa), jnp.exp(sb), jnp.exp(sd)
        d = (jnp.sum(wa, axis=1, keepdims=True)
             + jnp.sum(wb_, axis=1, keepdims=True)
             + jnp.sum(wd, axis=1, keepdims=True))
        c = (jnp.dot(wa.astype(bf), vb0, preferred_element_type=jnp.float32)
             + jnp.dot(wb_.astype(bf), v_main, preferred_element_type=jnp.float32)
             + jnp.dot(wd.astype(bf), v_diag, preferred_element_type=jnp.float32))
        return (c / d).astype(bf)

    def gather_blocks(a, blocks):
        return jnp.concatenate([a[b * BLK:(b + 1) * BLK] for b in blocks],
                               axis=0)

    def compute_group(g, wq_at, wo_at):
        def head_body(h, carry):
            gh = g * H_PER + h
            qh = jnp.dot(x_ref[...], wq_at(h),
                         preferred_element_type=jnp.float32
                         ).astype(bf)
            kh = k_ref[gh]
            vh = v_ref[gh]
            k0s, k1s, k2s = (gather_blocks(kh, b) for b in (_B0, _B1, _B2))
            v0s, v1s, v2s = (gather_blocks(vh, b) for b in (_B0, _B1, _B2))
            kb0, vb0 = kh[0:BLK], vh[0:BLK]
            s0 = lax.dot_general(gather_blocks(qh, _B0), k0s,
                                 (((1,), (1,)), ((), ())),
                                 preferred_element_type=jnp.float32)
            w0 = jnp.exp(s0)
            d0 = jnp.sum(w0, axis=1, keepdims=True)
            c0 = jnp.dot(w0.astype(bf), v0s,
                         preferred_element_type=jnp.float32)
            ctx0 = (c0 / d0).astype(bf)
            ctx1 = piece(gather_blocks(qh, _B1), k2s, v2s, kb0, vb0, k1s, v1s)
            ctx2 = piece(gather_blocks(qh, _B2), k1s, v1s, kb0, vb0, k2s, v2s)
            ctx = jnp.concatenate([ctx0, ctx1, ctx2], axis=0)
            ctx = gather_blocks(ctx, _INV_ORDER)
            out_ref[...] = out_ref[...] + jnp.dot(
                ctx, wo_at(h), preferred_element_type=jnp.float32)
            return carry
        lax.fori_loop(0, H_PER, head_body, 0)

    out_ref[...] = jnp.zeros((SQ, D_MODEL), jnp.float32)
    compute_group(my, lambda h: wq_ref[h], lambda h: wo_ref[h])

    for i in (0, 1):
        rq[i].wait_send()
        ro[i].wait_send()
    rq[2].start()
    ro[2].start()

    for slot, off in ((0, N_DEV - 1), (1, 1), (2, 2)):
        rq[slot].wait_recv()
        ro[slot].wait_recv()
        g = lax.rem(my + off, N_DEV)
        compute_group(g,
                      lambda h: commq[slot, h],
                      lambda h: commo[slot, h])

    rq[2].wait_send()
    ro[2].wait_send()


def kernel(x, Wq, K_ext, V_ext, Wo):
    my = lax.axis_index("i")
    bf = jnp.bfloat16
    xb = x[0].astype(bf)
    wq3 = jnp.swapaxes(
        (Wq * SCALE).reshape(D_MODEL, H_PER, DH), 0, 1).astype(bf)
    wo3 = Wo.reshape(H_PER, DH, D_MODEL).astype(bf)
    kb = jnp.swapaxes(
        lax.dynamic_index_in_dim(K_ext, my, 0, keepdims=False), 0, 1).astype(bf)
    vb = jnp.swapaxes(
        lax.dynamic_index_in_dim(V_ext, my, 0, keepdims=False), 0, 1).astype(bf)

    out = pl.pallas_call(
        _body,
        out_shape=jax.ShapeDtypeStruct((SQ, D_MODEL), jnp.float32),
        in_specs=[
            pl.BlockSpec(memory_space=pltpu.VMEM),
            pl.BlockSpec(memory_space=pltpu.VMEM),
            pl.BlockSpec(memory_space=pltpu.VMEM),
            pl.BlockSpec(memory_space=pltpu.VMEM),
            pl.BlockSpec(memory_space=pltpu.VMEM),
        ],
        out_specs=pl.BlockSpec(memory_space=pltpu.VMEM),
        scratch_shapes=[
            pltpu.VMEM((N_DEV - 1, H_PER, D_MODEL, DH), bf),
            pltpu.VMEM((N_DEV - 1, H_PER, DH, D_MODEL), bf),
            pltpu.VMEM((R1 - R0, R1 - R0), jnp.float32),
            pltpu.SemaphoreType.DMA((N_DEV - 1,)),
            pltpu.SemaphoreType.DMA((N_DEV - 1,)),
            pltpu.SemaphoreType.DMA((N_DEV - 1,)),
            pltpu.SemaphoreType.DMA((N_DEV - 1,)),
        ],
        compiler_params=pltpu.CompilerParams(collective_id=0),
    )(xb, wq3, wo3, kb, vb)
    return out[None]


# device time: 171296 ns/iter; 1.0742x vs baseline; 1.0092x over previous
import numpy as np

import jax
import jax.numpy as jnp
from jax import lax
from jax.experimental import pallas as pl
from jax.experimental.pallas import tpu as pltpu

N_DEV = 4
SQ = 1024
SKV = 1024
D_MODEL = 1024
H_PER = 8
DH = 128
BLK = 64
SCALE = 0.08838834764831843

_B0 = [0, 3, 6, 9, 12, 15]
_B1 = [1, 4, 7, 10, 13]
_B2 = [2, 5, 8, 11, 14]
_PBLOCKS = _B0 + _B1 + _B2
_INV_ORDER = [_PBLOCKS.index(b) for b in range(16)]
R0 = 6 * BLK
R1 = 11 * BLK


def _body(x_ref, wq_ref, wo_ref, k_ref, v_ref, out_ref,
          commq, commo, dbias_ref, sendq, recvq, sendo, recvo):
    my = lax.axis_index("i")
    right = lax.rem(my + 1, N_DEV)
    left = lax.rem(my + N_DEV - 1, N_DEV)
    diag = lax.rem(my + 2, N_DEV)

    barrier_sem = pltpu.get_barrier_semaphore()
    for nbr in (left, right, diag):
        pl.semaphore_signal(
            barrier_sem, inc=1,
            device_id=(nbr,), device_id_type=pl.DeviceIdType.MESH,
        )
    pl.semaphore_wait(barrier_sem, 3)

    rb = lax.broadcasted_iota(jnp.int32, (R1 - R0, R1 - R0), 0) // BLK
    cb = lax.broadcasted_iota(jnp.int32, (R1 - R0, R1 - R0), 1) // BLK
    dbias_ref[...] = jnp.where(rb == cb, jnp.float32(0.0), jnp.float32(-1e9))

    def mk(src, comm, slot, ssems, rsems, dev):
        return pltpu.make_async_remote_copy(
            src_ref=src, dst_ref=comm.at[slot],
            send_sem=ssems.at[slot], recv_sem=rsems.at[slot],
            device_id=(dev,), device_id_type=pl.DeviceIdType.MESH,
        )

    rq = [mk(wq_ref, commq, 0, sendq, recvq, right),
          mk(wq_ref, commq, 1, sendq, recvq, left),
          mk(wq_ref, commq, 2, sendq, recvq, diag)]
    ro = [mk(wo_ref, commo, 0, sendo, recvo, right),
          mk(wo_ref, commo, 1, sendo, recvo, left),
          mk(wo_ref, commo, 2, sendo, recvo, diag)]

    for i in (0, 1):
        rq[i].start()
        ro[i].start()

    bf = jnp.bfloat16

    def piece(q, k_main, v_main, kb0, vb0, k_diag, v_diag):
        sa = lax.dot_general(q, kb0, (((1,), (1,)), ((), ())),
                             preferred_element_type=jnp.float32)
        sb = lax.dot_general(q, k_main, (((1,), (1,)), ((), ())),
                             preferred_element_type=jnp.float32)
        sd = lax.dot_general(q, k_diag, (((1,), (1,)), ((), ())),
                             preferred_element_type=jnp.float32) + dbias_ref[...]
        wa, wb_, wd = jnp.exp(sa), jnp.exp(sb), jnp.exp(sd)
        d = (jnp.sum(wa, axis=1, keepdims=True)
             + jnp.sum(wb_, axis=1, keepdims=True)
             + jnp.sum(wd, axis=1, keepdims=True))
        c = (jnp.dot(wa.astype(bf), vb0, preferred_element_type=jnp.float32)
             + jnp.dot(wb_.astype(bf), v_main, preferred_element_type=jnp.float32)
             + jnp.dot(wd.astype(bf), v_diag, preferred_element_type=jnp.float32))
        return (c / d).astype(bf)

    def gather_blocks(a, blocks):
        return jnp.concatenate([a[b * BLK:(b + 1) * BLK] for b in blocks],
                               axis=0)

    def compute_group(g, wq_at, wo_at):
        def head_body(h):
            gh = g * H_PER + h
            qh = jnp.dot(x_ref[...], wq_at(h),
                         preferred_element_type=jnp.float32
                         ).astype(bf)
            kh = k_ref[gh]
            vh = v_ref[gh]
            k0s, k1s, k2s = (gather_blocks(kh, b) for b in (_B0, _B1, _B2))
            v0s, v1s, v2s = (gather_blocks(vh, b) for b in (_B0, _B1, _B2))
            kb0, vb0 = kh[0:BLK], vh[0:BLK]
            s0 = lax.dot_general(gather_blocks(qh, _B0), k0s,
                                 (((1,), (1,)), ((), ())),
                                 preferred_element_type=jnp.float32)
            w0 = jnp.exp(s0)
            d0 = jnp.sum(w0, axis=1, keepdims=True)
            c0 = jnp.dot(w0.astype(bf), v0s,
                         preferred_element_type=jnp.float32)
            ctx0 = (c0 / d0).astype(bf)
            ctx1 = piece(gather_blocks(qh, _B1), k2s, v2s, kb0, vb0, k1s, v1s)
            ctx2 = piece(gather_blocks(qh, _B2), k1s, v1s, kb0, vb0, k2s, v2s)
            ctx = jnp.concatenate([ctx0, ctx1, ctx2], axis=0)
            ctx = gather_blocks(ctx, _INV_ORDER)
            out_ref[...] = out_ref[...] + jnp.dot(
                ctx, wo_at(h), preferred_element_type=jnp.float32)
        for h in range(H_PER):
            head_body(h)

    out_ref[...] = jnp.zeros((SQ, D_MODEL), jnp.float32)
    compute_group(my, lambda h: wq_ref[h], lambda h: wo_ref[h])

    for i in (0, 1):
        rq[i].wait_send()
        ro[i].wait_send()
    rq[2].start()
    ro[2].start()

    for slot, off in ((0, N_DEV - 1), (1, 1), (2, 2)):
        rq[slot].wait_recv()
        ro[slot].wait_recv()
        g = lax.rem(my + off, N_DEV)
        compute_group(g,
                      lambda h: commq[slot, h],
                      lambda h: commo[slot, h])

    rq[2].wait_send()
    ro[2].wait_send()


def kernel(x, Wq, K_ext, V_ext, Wo):
    my = lax.axis_index("i")
    bf = jnp.bfloat16
    xb = x[0].astype(bf)
    wq3 = jnp.swapaxes(
        (Wq * SCALE).reshape(D_MODEL, H_PER, DH), 0, 1).astype(bf)
    wo3 = Wo.reshape(H_PER, DH, D_MODEL).astype(bf)
    kb = jnp.swapaxes(
        lax.dynamic_index_in_dim(K_ext, my, 0, keepdims=False), 0, 1).astype(bf)
    vb = jnp.swapaxes(
        lax.dynamic_index_in_dim(V_ext, my, 0, keepdims=False), 0, 1).astype(bf)

    out = pl.pallas_call(
        _body,
        out_shape=jax.ShapeDtypeStruct((SQ, D_MODEL), jnp.float32),
        in_specs=[
            pl.BlockSpec(memory_space=pltpu.VMEM),
            pl.BlockSpec(memory_space=pltpu.VMEM),
            pl.BlockSpec(memory_space=pltpu.VMEM),
            pl.BlockSpec(memory_space=pltpu.VMEM),
            pl.BlockSpec(memory_space=pltpu.VMEM),
        ],
        out_specs=pl.BlockSpec(memory_space=pltpu.VMEM),
        scratch_shapes=[
            pltpu.VMEM((N_DEV - 1, H_PER, D_MODEL, DH), bf),
            pltpu.VMEM((N_DEV - 1, H_PER, DH, D_MODEL), bf),
            pltpu.VMEM((R1 - R0, R1 - R0), jnp.float32),
            pltpu.SemaphoreType.DMA((N_DEV - 1,)),
            pltpu.SemaphoreType.DMA((N_DEV - 1,)),
            pltpu.SemaphoreType.DMA((N_DEV - 1,)),
            pltpu.SemaphoreType.DMA((N_DEV - 1,)),
        ],
        compiler_params=pltpu.CompilerParams(collective_id=0),
    )(xb, wq3, wo3, kb, vb)
    return out[None]


# device time: 162474 ns/iter; 1.1325x vs baseline; 1.0543x over previous
import numpy as np

import jax
import jax.numpy as jnp
from jax import lax
from jax.experimental import pallas as pl
from jax.experimental.pallas import tpu as pltpu

N_DEV = 4
SQ = 1024
SKV = 1024
D_MODEL = 1024
H_PER = 8
DH = 128
BLK = 64
SCALE = 0.08838834764831843

_B0 = [0, 3, 6, 9, 12, 15]
_B1 = [1, 4, 7, 10, 13]
_B2 = [2, 5, 8, 11, 14]
_PBLOCKS = _B0 + _B1 + _B2
_INV_ORDER = [_PBLOCKS.index(b) for b in range(16)]
R0 = 6 * BLK
R1 = 11 * BLK


def _body(x_ref, wq_ref, wo_ref, k_ref, v_ref, out_ref,
          commq, commo, dbias_ref, sendq, recvq, sendo, recvo):
    my = lax.axis_index("i")
    right = lax.rem(my + 1, N_DEV)
    left = lax.rem(my + N_DEV - 1, N_DEV)
    diag = lax.rem(my + 2, N_DEV)

    barrier_sem = pltpu.get_barrier_semaphore()
    for nbr in (left, right, diag):
        pl.semaphore_signal(
            barrier_sem, inc=1,
            device_id=(nbr,), device_id_type=pl.DeviceIdType.MESH,
        )
    pl.semaphore_wait(barrier_sem, 3)

    rb = lax.broadcasted_iota(jnp.int32, (R1 - R0, R1 - R0), 0) // BLK
    cb = lax.broadcasted_iota(jnp.int32, (R1 - R0, R1 - R0), 1) // BLK
    dbias_ref[...] = jnp.where(rb == cb, jnp.float32(0.0), jnp.float32(-1e9))

    def mk(src, comm, slot, ssems, rsems, dev):
        return pltpu.make_async_remote_copy(
            src_ref=src, dst_ref=comm.at[slot],
            send_sem=ssems.at[slot], recv_sem=rsems.at[slot],
            device_id=(dev,), device_id_type=pl.DeviceIdType.MESH,
        )

    rq = [mk(wq_ref, commq, 0, sendq, recvq, right),
          mk(wq_ref, commq, 1, sendq, recvq, left),
          mk(wq_ref, commq, 2, sendq, recvq, diag)]
    ro = [mk(wo_ref, commo, 0, sendo, recvo, right),
          mk(wo_ref, commo, 1, sendo, recvo, left),
          mk(wo_ref, commo, 2, sendo, recvo, diag)]

    for i in (0, 1):
        rq[i].start()
        ro[i].start()

    bf = jnp.bfloat16

    def piece(q, k_main, v_main, kb0, vb0, k_diag, v_diag):
        sa = lax.dot_general(q, kb0, (((1,), (1,)), ((), ())),
                             preferred_element_type=jnp.float32)
        sb = lax.dot_general(q, k_main, (((1,), (1,)), ((), ())),
                             preferred_element_type=jnp.float32)
        sd = lax.dot_general(q, k_diag, (((1,), (1,)), ((), ())),
                             preferred_element_type=jnp.float32) + dbias_ref[...]
        wa, wb_, wd = jnp.exp(sa), jnp.exp(sb), jnp.exp(sd)
        d = (jnp.sum(wa, axis=1, keepdims=True)
             + jnp.sum(wb_, axis=1, keepdims=True)
             + jnp.sum(wd, axis=1, keepdims=True))
        c = (jnp.dot(wa.astype(bf), vb0, preferred_element_type=jnp.float32)
             + jnp.dot(wb_.astype(bf), v_main, preferred_element_type=jnp.float32)
             + jnp.dot(wd.astype(bf), v_diag, preferred_element_type=jnp.float32))
        return (c / d).astype(bf)

    def gather_blocks(a, blocks):
        return jnp.concatenate([a[b * BLK:(b + 1) * BLK] for b in blocks],
                               axis=0)

    def compute_group(g, wq_at, wo_2d, accumulate):
        ctxs = []
        for h in range(H_PER):
            gh = g * H_PER + h
            qh = jnp.dot(x_ref[...], wq_at(h),
                         preferred_element_type=jnp.float32
                         ).astype(bf)
            kh = k_ref[gh]
            vh = v_ref[gh]
            k0s, k1s, k2s = (gather_blocks(kh, b) for b in (_B0, _B1, _B2))
            v0s, v1s, v2s = (gather_blocks(vh, b) for b in (_B0, _B1, _B2))
            kb0, vb0 = kh[0:BLK], vh[0:BLK]
            s0 = lax.dot_general(gather_blocks(qh, _B0), k0s,
                                 (((1,), (1,)), ((), ())),
                                 preferred_element_type=jnp.float32)
            w0 = jnp.exp(s0)
            d0 = jnp.sum(w0, axis=1, keepdims=True)
            c0 = jnp.dot(w0.astype(bf), v0s,
                         preferred_element_type=jnp.float32)
            ctx0 = (c0 / d0).astype(bf)
            ctx1 = piece(gather_blocks(qh, _B1), k2s, v2s, kb0, vb0, k1s, v1s)
            ctx2 = piece(gather_blocks(qh, _B2), k1s, v1s, kb0, vb0, k2s, v2s)
            ctx = jnp.concatenate([ctx0, ctx1, ctx2], axis=0)
            ctxs.append(gather_blocks(ctx, _INV_ORDER))
        ctx_all = jnp.concatenate(ctxs, axis=1)
        contrib = jnp.dot(ctx_all, wo_2d, preferred_element_type=jnp.float32)
        if accumulate:
            out_ref[...] = out_ref[...] + contrib
        else:
            out_ref[...] = contrib

    compute_group(my, lambda h: wq_ref[h], wo_ref[...], accumulate=False)

    for i in (0, 1):
        rq[i].wait_send()
        ro[i].wait_send()
    rq[2].start()
    ro[2].start()

    for slot, off in ((0, N_DEV - 1), (1, 1), (2, 2)):
        rq[slot].wait_recv()
        ro[slot].wait_recv()
        g = lax.rem(my + off, N_DEV)
        compute_group(g,
                      lambda h: commq[slot, h],
                      commo[slot],
                      accumulate=True)

    rq[2].wait_send()
    ro[2].wait_send()


def kernel(x, Wq, K_ext, V_ext, Wo):
    my = lax.axis_index("i")
    bf = jnp.bfloat16
    xb = x[0].astype(bf)
    wq3 = jnp.swapaxes(
        (Wq * SCALE).reshape(D_MODEL, H_PER, DH), 0, 1).astype(bf)
    wo2 = Wo.astype(bf)
    kb = jnp.swapaxes(
        lax.dynamic_index_in_dim(K_ext, my, 0, keepdims=False), 0, 1).astype(bf)
    vb = jnp.swapaxes(
        lax.dynamic_index_in_dim(V_ext, my, 0, keepdims=False), 0, 1).astype(bf)

    out = pl.pallas_call(
        _body,
        out_shape=jax.ShapeDtypeStruct((SQ, D_MODEL), jnp.float32),
        in_specs=[
            pl.BlockSpec(memory_space=pltpu.VMEM),
            pl.BlockSpec(memory_space=pltpu.VMEM),
            pl.BlockSpec(memory_space=pltpu.VMEM),
            pl.BlockSpec(memory_space=pltpu.VMEM),
            pl.BlockSpec(memory_space=pltpu.VMEM),
        ],
        out_specs=pl.BlockSpec(memory_space=pltpu.VMEM),
        scratch_shapes=[
            pltpu.VMEM((N_DEV - 1, H_PER, D_MODEL, DH), bf),
            pltpu.VMEM((N_DEV - 1, D_MODEL, D_MODEL), bf),
            pltpu.VMEM((R1 - R0, R1 - R0), jnp.float32),
            pltpu.SemaphoreType.DMA((N_DEV - 1,)),
            pltpu.SemaphoreType.DMA((N_DEV - 1,)),
            pltpu.SemaphoreType.DMA((N_DEV - 1,)),
            pltpu.SemaphoreType.DMA((N_DEV - 1,)),
        ],
        compiler_params=pltpu.CompilerParams(collective_id=0),
    )(xb, wq3, wo2, kb, vb)
    return out[None]


# device time: 160742 ns/iter; 1.1447x vs baseline; 1.0108x over previous
import numpy as np

import jax
import jax.numpy as jnp
from jax import lax
from jax.experimental import pallas as pl
from jax.experimental.pallas import tpu as pltpu

N_DEV = 4
SQ = 1024
SKV = 1024
D_MODEL = 1024
H_PER = 8
DH = 128
BLK = 64
SCALE = 0.08838834764831843

_B0 = [0, 3, 6, 9, 12, 15]
_B1 = [1, 4, 7, 10, 13]
_B2 = [2, 5, 8, 11, 14]
_PBLOCKS = _B0 + _B1 + _B2
_INV_ORDER = [_PBLOCKS.index(b) for b in range(16)]
R0 = 6 * BLK
R1 = 11 * BLK


def _body(x_ref, wq_ref, wo_ref, k_ref, v_ref, out_ref,
          commq, commo, dbias_ref, xp_ref, sendq, recvq, sendo, recvo):
    my = lax.axis_index("i")
    right = lax.rem(my + 1, N_DEV)
    left = lax.rem(my + N_DEV - 1, N_DEV)
    diag = lax.rem(my + 2, N_DEV)

    barrier_sem = pltpu.get_barrier_semaphore()
    for nbr in (left, right, diag):
        pl.semaphore_signal(
            barrier_sem, inc=1,
            device_id=(nbr,), device_id_type=pl.DeviceIdType.MESH,
        )
    pl.semaphore_wait(barrier_sem, 3)

    rb = lax.broadcasted_iota(jnp.int32, (R1 - R0, R1 - R0), 0) // BLK
    cb = lax.broadcasted_iota(jnp.int32, (R1 - R0, R1 - R0), 1) // BLK
    dbias_ref[...] = jnp.where(rb == cb, jnp.float32(0.0), jnp.float32(-1e9))

    def mk(src, comm, slot, ssems, rsems, dev):
        return pltpu.make_async_remote_copy(
            src_ref=src, dst_ref=comm.at[slot],
            send_sem=ssems.at[slot], recv_sem=rsems.at[slot],
            device_id=(dev,), device_id_type=pl.DeviceIdType.MESH,
        )

    rq = [mk(wq_ref, commq, 0, sendq, recvq, right),
          mk(wq_ref, commq, 1, sendq, recvq, left),
          mk(wq_ref, commq, 2, sendq, recvq, diag)]
    ro = [mk(wo_ref, commo, 0, sendo, recvo, right),
          mk(wo_ref, commo, 1, sendo, recvo, left),
          mk(wo_ref, commo, 2, sendo, recvo, diag)]

    for i in (0, 1):
        rq[i].start()
        ro[i].start()

    bf = jnp.bfloat16

    def piece(q, k_main, v_main, kb0, vb0, k_diag, v_diag):
        sa = lax.dot_general(q, kb0, (((1,), (1,)), ((), ())),
                             preferred_element_type=jnp.float32)
        sb = lax.dot_general(q, k_main, (((1,), (1,)), ((), ())),
                             preferred_element_type=jnp.float32)
        sd = lax.dot_general(q, k_diag, (((1,), (1,)), ((), ())),
                             preferred_element_type=jnp.float32) + dbias_ref[...]
        wa, wb_, wd = jnp.exp(sa), jnp.exp(sb), jnp.exp(sd)
        d = (jnp.sum(wa, axis=1, keepdims=True)
             + jnp.sum(wb_, axis=1, keepdims=True)
             + jnp.sum(wd, axis=1, keepdims=True))
        c = (jnp.dot(wa.astype(bf), vb0, preferred_element_type=jnp.float32)
             + jnp.dot(wb_.astype(bf), v_main, preferred_element_type=jnp.float32)
             + jnp.dot(wd.astype(bf), v_diag, preferred_element_type=jnp.float32))
        return (c / d).astype(bf)

    def gather_blocks(a, blocks):
        return jnp.concatenate([a[b * BLK:(b + 1) * BLK] for b in blocks],
                               axis=0)

    xp_ref[...] = gather_blocks(x_ref[...], _PBLOCKS)

    def permute_kv(h, carry):
        k_ref[h] = gather_blocks(k_ref[h], _PBLOCKS)
        v_ref[h] = gather_blocks(v_ref[h], _PBLOCKS)
        return carry
    lax.fori_loop(0, N_DEV * H_PER, permute_kv, 0)

    def compute_group(g, wq_at, wo_2d, accumulate):
        ctxs = []
        for h in range(H_PER):
            gh = g * H_PER + h
            qh = jnp.dot(xp_ref[...], wq_at(h),
                         preferred_element_type=jnp.float32
                         ).astype(bf)
            kh = k_ref[gh]
            vh = v_ref[gh]
            kb0, vb0 = kh[0:BLK], vh[0:BLK]
            s0 = lax.dot_general(qh[0:R0], kh[0:R0],
                                 (((1,), (1,)), ((), ())),
                                 preferred_element_type=jnp.float32)
            w0 = jnp.exp(s0)
            d0 = jnp.sum(w0, axis=1, keepdims=True)
            c0 = jnp.dot(w0.astype(bf), vh[0:R0],
                         preferred_element_type=jnp.float32)
            ctx0 = (c0 / d0).astype(bf)
            ctx1 = piece(qh[R0:R1], kh[R1:SKV], vh[R1:SKV],
                         kb0, vb0, kh[R0:R1], vh[R0:R1])
            ctx2 = piece(qh[R1:SKV], kh[R0:R1], vh[R0:R1],
                         kb0, vb0, kh[R1:SKV], vh[R1:SKV])
            ctxs.append(jnp.concatenate([ctx0, ctx1, ctx2], axis=0))
        ctx_all = jnp.concatenate(ctxs, axis=1)
        contrib = jnp.dot(ctx_all, wo_2d, preferred_element_type=jnp.float32)
        if accumulate:
            out_ref[...] = out_ref[...] + contrib
        else:
            out_ref[...] = contrib

    compute_group(my, lambda h: wq_ref[h], wo_ref[...], accumulate=False)

    for i in (0, 1):
        rq[i].wait_send()
        ro[i].wait_send()
    rq[2].start()
    ro[2].start()

    for slot, off in ((0, N_DEV - 1), (1, 1), (2, 2)):
        rq[slot].wait_recv()
        ro[slot].wait_recv()
        g = lax.rem(my + off, N_DEV)
        compute_group(g,
                      lambda h: commq[slot, h],
                      commo[slot],
                      accumulate=True)

    out_ref[...] = gather_blocks(out_ref[...], _INV_ORDER)

    rq[2].wait_send()
    ro[2].wait_send()


def kernel(x, Wq, K_ext, V_ext, Wo):
    my = lax.axis_index("i")
    bf = jnp.bfloat16
    xb = x[0].astype(bf)
    wq3 = jnp.swapaxes(
        (Wq * SCALE).reshape(D_MODEL, H_PER, DH), 0, 1).astype(bf)
    wo2 = Wo.astype(bf)
    kb = jnp.swapaxes(
        lax.dynamic_index_in_dim(K_ext, my, 0, keepdims=False), 0, 1).astype(bf)
    vb = jnp.swapaxes(
        lax.dynamic_index_in_dim(V_ext, my, 0, keepdims=False), 0, 1).astype(bf)

    out = pl.pallas_call(
        _body,
        out_shape=jax.ShapeDtypeStruct((SQ, D_MODEL), jnp.float32),
        in_specs=[
            pl.BlockSpec(memory_space=pltpu.VMEM),
            pl.BlockSpec(memory_space=pltpu.VMEM),
            pl.BlockSpec(memory_space=pltpu.VMEM),
            pl.BlockSpec(memory_space=pltpu.VMEM),
            pl.BlockSpec(memory_space=pltpu.VMEM),
        ],
        out_specs=pl.BlockSpec(memory_space=pltpu.VMEM),
        scratch_shapes=[
            pltpu.VMEM((N_DEV - 1, H_PER, D_MODEL, DH), bf),
            pltpu.VMEM((N_DEV - 1, D_MODEL, D_MODEL), bf),
            pltpu.VMEM((R1 - R0, R1 - R0), jnp.float32),
            pltpu.VMEM((SQ, D_MODEL), bf),
            pltpu.SemaphoreType.DMA((N_DEV - 1,)),
            pltpu.SemaphoreType.DMA((N_DEV - 1,)),
            pltpu.SemaphoreType.DMA((N_DEV - 1,)),
            pltpu.SemaphoreType.DMA((N_DEV - 1,)),
        ],
        compiler_params=pltpu.CompilerParams(collective_id=0),
    )(xb, wq3, wo2, kb, vb)
    return out[None]


# device time: 159758 ns/iter; 1.1518x vs baseline; 1.0062x over previous
import numpy as np

import jax
import jax.numpy as jnp
from jax import lax
from jax.experimental import pallas as pl
from jax.experimental.pallas import tpu as pltpu

N_DEV = 4
SQ = 1024
SKV = 1024
D_MODEL = 1024
H_PER = 8
DH = 128
BLK = 64
SCALE = 0.08838834764831843

_B0 = [0, 3, 6, 9, 12, 15]
_B1 = [1, 4, 7, 10, 13]
_B2 = [2, 5, 8, 11, 14]
_PBLOCKS = _B0 + _B1 + _B2
_INV_ORDER = [_PBLOCKS.index(b) for b in range(16)]
R0 = 6 * BLK
R1 = 11 * BLK


def _body(x_ref, wq_ref, wo_ref, k_ref, v_ref, out_ref,
          commq, commo, dbias_ref, xp_ref, sendq, recvq, sendo, recvo):
    my = lax.axis_index("i")
    right = lax.rem(my + 1, N_DEV)
    left = lax.rem(my + N_DEV - 1, N_DEV)
    diag = lax.rem(my + 2, N_DEV)

    barrier_sem = pltpu.get_barrier_semaphore()
    for nbr in (left, right, diag):
        pl.semaphore_signal(
            barrier_sem, inc=1,
            device_id=(nbr,), device_id_type=pl.DeviceIdType.MESH,
        )
    pl.semaphore_wait(barrier_sem, 3)

    rb = lax.broadcasted_iota(jnp.int32, (R1 - R0, R1 - R0), 0) // BLK
    cb = lax.broadcasted_iota(jnp.int32, (R1 - R0, R1 - R0), 1) // BLK
    dbias_ref[...] = jnp.where(rb == cb, jnp.float32(0.0), jnp.float32(-1e9))

    def mk(src, comm, slot, ssems, rsems, dev):
        return pltpu.make_async_remote_copy(
            src_ref=src, dst_ref=comm.at[slot],
            send_sem=ssems.at[slot], recv_sem=rsems.at[slot],
            device_id=(dev,), device_id_type=pl.DeviceIdType.MESH,
        )

    rq = [mk(wq_ref, commq, 0, sendq, recvq, right),
          mk(wq_ref, commq, 1, sendq, recvq, left),
          mk(wq_ref, commq, 2, sendq, recvq, diag)]
    ro = [mk(wo_ref, commo, 0, sendo, recvo, right),
          mk(wo_ref, commo, 1, sendo, recvo, left),
          mk(wo_ref, commo, 2, sendo, recvo, diag)]

    rq[0].start()
    rq[1].start()

    bf = jnp.bfloat16

    def piece(q, k_main, v_main, kb0, vb0, k_diag, v_diag):
        sa = lax.dot_general(q, kb0, (((1,), (1,)), ((), ())),
                             preferred_element_type=jnp.float32)
        sb = lax.dot_general(q, k_main, (((1,), (1,)), ((), ())),
                             preferred_element_type=jnp.float32)
        sd = lax.dot_general(q, k_diag, (((1,), (1,)), ((), ())),
                             preferred_element_type=jnp.float32) + dbias_ref[...]
        wa, wb_, wd = jnp.exp(sa), jnp.exp(sb), jnp.exp(sd)
        d = (jnp.sum(wa, axis=1, keepdims=True)
             + jnp.sum(wb_, axis=1, keepdims=True)
             + jnp.sum(wd, axis=1, keepdims=True))
        c = (jnp.dot(wa.astype(bf), vb0, preferred_element_type=jnp.float32)
             + jnp.dot(wb_.astype(bf), v_main, preferred_element_type=jnp.float32)
             + jnp.dot(wd.astype(bf), v_diag, preferred_element_type=jnp.float32))
        return (c / d).astype(bf)

    def gather_blocks(a, blocks):
        return jnp.concatenate([a[b * BLK:(b + 1) * BLK] for b in blocks],
                               axis=0)

    xp_ref[...] = gather_blocks(x_ref[...], _PBLOCKS)

    def permute_kv(h, carry):
        k_ref[h] = gather_blocks(k_ref[h], _PBLOCKS)
        v_ref[h] = gather_blocks(v_ref[h], _PBLOCKS)
        return carry
    lax.fori_loop(0, N_DEV * H_PER, permute_kv, 0)

    def compute_group(g, wq_at, wo_thunk, accumulate):
        ctxs = []
        for h in range(H_PER):
            gh = g * H_PER + h
            qh = jnp.dot(xp_ref[...], wq_at(h),
                         preferred_element_type=jnp.float32
                         ).astype(bf)
            kh = k_ref[gh]
            vh = v_ref[gh]
            kb0, vb0 = kh[0:BLK], vh[0:BLK]
            s0 = lax.dot_general(qh[0:R0], kh[0:R0],
                                 (((1,), (1,)), ((), ())),
                                 preferred_element_type=jnp.float32)
            w0 = jnp.exp(s0)
            d0 = jnp.sum(w0, axis=1, keepdims=True)
            c0 = jnp.dot(w0.astype(bf), vh[0:R0],
                         preferred_element_type=jnp.float32)
            ctx0 = (c0 / d0).astype(bf)
            ctx1 = piece(qh[R0:R1], kh[R1:SKV], vh[R1:SKV],
                         kb0, vb0, kh[R0:R1], vh[R0:R1])
            ctx2 = piece(qh[R1:SKV], kh[R0:R1], vh[R0:R1],
                         kb0, vb0, kh[R1:SKV], vh[R1:SKV])
            ctxs.append(jnp.concatenate([ctx0, ctx1, ctx2], axis=0))
        ctx_all = jnp.concatenate(ctxs, axis=1)
        contrib = jnp.dot(ctx_all, wo_thunk(),
                          preferred_element_type=jnp.float32)
        if accumulate:
            out_ref[...] = out_ref[...] + contrib
        else:
            out_ref[...] = contrib

    def wo_slot(i):
        def thunk():
            ro[i].wait_recv()
            return commo[i]
        return thunk

    rq[0].wait_send()
    rq[1].wait_send()
    ro[0].start()
    ro[1].start()
    compute_group(my, lambda h: wq_ref[h], lambda: wo_ref[...],
                  accumulate=False)

    ro[0].wait_send()
    ro[1].wait_send()
    rq[2].start()

    rq[0].wait_recv()
    compute_group(lax.rem(my + N_DEV - 1, N_DEV),
                  lambda h: commq[0, h], wo_slot(0), accumulate=True)

    rq[2].wait_send()
    ro[2].start()

    rq[1].wait_recv()
    compute_group(lax.rem(my + 1, N_DEV),
                  lambda h: commq[1, h], wo_slot(1), accumulate=True)

    rq[2].wait_recv()
    compute_group(lax.rem(my + 2, N_DEV),
                  lambda h: commq[2, h], wo_slot(2), accumulate=True)

    out_ref[...] = gather_blocks(out_ref[...], _INV_ORDER)

    ro[2].wait_send()


def kernel(x, Wq, K_ext, V_ext, Wo):
    my = lax.axis_index("i")
    bf = jnp.bfloat16
    xb = x[0].astype(bf)
    wq3 = jnp.swapaxes(
        (Wq * SCALE).reshape(D_MODEL, H_PER, DH), 0, 1).astype(bf)
    wo2 = Wo.astype(bf)
    kb = jnp.swapaxes(
        lax.dynamic_index_in_dim(K_ext, my, 0, keepdims=False), 0, 1).astype(bf)
    vb = jnp.swapaxes(
        lax.dynamic_index_in_dim(V_ext, my, 0, keepdims=False), 0, 1).astype(bf)

    out = pl.pallas_call(
        _body,
        out_shape=jax.ShapeDtypeStruct((SQ, D_MODEL), jnp.float32),
        in_specs=[
            pl.BlockSpec(memory_space=pltpu.VMEM),
            pl.BlockSpec(memory_space=pltpu.VMEM),
            pl.BlockSpec(memory_space=pltpu.VMEM),
            pl.BlockSpec(memory_space=pltpu.VMEM),
            pl.BlockSpec(memory_space=pltpu.VMEM),
        ],
        out_specs=pl.BlockSpec(memory_space=pltpu.VMEM),
        scratch_shapes=[
            pltpu.VMEM((N_DEV - 1, H_PER, D_MODEL, DH), bf),
            pltpu.VMEM((N_DEV - 1, D_MODEL, D_MODEL), bf),
            pltpu.VMEM((R1 - R0, R1 - R0), jnp.float32),
            pltpu.VMEM((SQ, D_MODEL), bf),
            pltpu.SemaphoreType.DMA((N_DEV - 1,)),
            pltpu.SemaphoreType.DMA((N_DEV - 1,)),
            pltpu.SemaphoreType.DMA((N_DEV - 1,)),
            pltpu.SemaphoreType.DMA((N_DEV - 1,)),
        ],
        compiler_params=pltpu.CompilerParams(collective_id=0),
    )(xb, wq3, wo2, kb, vb)
    return out[None]


# device time: 141279 ns/iter; 1.3024x vs baseline; 1.1308x over previous
import numpy as np

import jax
import jax.numpy as jnp
from jax import lax
from jax.experimental import pallas as pl
from jax.experimental.pallas import tpu as pltpu

N_DEV = 4
SQ = 1024
SKV = 1024
D_MODEL = 1024
H_PER = 8
DH = 128
BLK = 64
SCALE = 0.08838834764831843

_B0 = [0, 3, 6, 9, 12, 15]
_B1 = [1, 4, 7, 10, 13]
_B2 = [2, 5, 8, 11, 14]
_PBLOCKS = _B0 + _B1 + _B2
_INV_ORDER = [_PBLOCKS.index(b) for b in range(16)]
R0 = 6 * BLK
R1 = 11 * BLK


def _body(x_ref, wq_ref, wo_ref, k_ref, v_ref, out_ref,
          commq, commo, dbias_ref, xp_ref, sendq, recvq, sendo, recvo):
    my = lax.axis_index("i")
    right = lax.rem(my + 1, N_DEV)
    left = lax.rem(my + N_DEV - 1, N_DEV)
    diag = lax.rem(my + 2, N_DEV)

    barrier_sem = pltpu.get_barrier_semaphore()
    for nbr in (left, right, diag):
        pl.semaphore_signal(
            barrier_sem, inc=1,
            device_id=(nbr,), device_id_type=pl.DeviceIdType.MESH,
        )
    pl.semaphore_wait(barrier_sem, 3)

    rb = lax.broadcasted_iota(jnp.int32, (R1 - R0, R1 - R0), 0) // BLK
    cb = lax.broadcasted_iota(jnp.int32, (R1 - R0, R1 - R0), 1) // BLK
    dbias_ref[...] = jnp.where(rb == cb, jnp.float32(0.0), jnp.float32(-1e9))

    def mk(src, comm, slot, ssems, rsems, dev):
        return pltpu.make_async_remote_copy(
            src_ref=src, dst_ref=comm.at[slot],
            send_sem=ssems.at[slot], recv_sem=rsems.at[slot],
            device_id=(dev,), device_id_type=pl.DeviceIdType.MESH,
        )

    rq = [mk(wq_ref, commq, 0, sendq, recvq, right),
          mk(wq_ref, commq, 1, sendq, recvq, left),
          mk(wq_ref, commq, 2, sendq, recvq, diag)]
    ro = [mk(wo_ref, commo, 0, sendo, recvo, right),
          mk(wo_ref, commo, 1, sendo, recvo, left),
          mk(wo_ref, commo, 2, sendo, recvo, diag)]

    rq[0].start()
    rq[1].start()

    bf = jnp.bfloat16

    def piece(q, k_main, v_main, kb0, vb0, k_diag, v_diag):
        sa = lax.dot_general(q, kb0, (((1,), (1,)), ((), ())),
                             preferred_element_type=jnp.float32)
        sb = lax.dot_general(q, k_main, (((1,), (1,)), ((), ())),
                             preferred_element_type=jnp.float32)
        sd = lax.dot_general(q, k_diag, (((1,), (1,)), ((), ())),
                             preferred_element_type=jnp.float32) + dbias_ref[...]
        wa, wb_, wd = jnp.exp(sa), jnp.exp(sb), jnp.exp(sd)
        d = (jnp.sum(wa, axis=1, keepdims=True)
             + jnp.sum(wb_, axis=1, keepdims=True)
             + jnp.sum(wd, axis=1, keepdims=True))
        c = (jnp.dot(wa.astype(bf), vb0, preferred_element_type=jnp.float32)
             + jnp.dot(wb_.astype(bf), v_main, preferred_element_type=jnp.float32)
             + jnp.dot(wd.astype(bf), v_diag, preferred_element_type=jnp.float32))
        return (c / d).astype(bf)

    def gather_blocks(a, blocks):
        return jnp.concatenate([a[b * BLK:(b + 1) * BLK] for b in blocks],
                               axis=0)

    xp_ref[...] = gather_blocks(x_ref[...], _PBLOCKS)

    def permute_kv(h, carry):
        k_ref[h] = gather_blocks(k_ref[h], _PBLOCKS)
        v_ref[h] = gather_blocks(v_ref[h], _PBLOCKS)
        return carry
    lax.fori_loop(0, N_DEV * H_PER, permute_kv, 0)

    def compute_group(g, wq_thunk, wo_thunk, accumulate):
        qg = jnp.dot(xp_ref[...], wq_thunk(),
                     preferred_element_type=jnp.float32).astype(bf)
        ctxs = []
        for h in range(H_PER):
            gh = g * H_PER + h
            qh = qg[:, h * DH:(h + 1) * DH]
            kh = k_ref[gh]
            vh = v_ref[gh]
            kb0, vb0 = kh[0:BLK], vh[0:BLK]
            s0 = lax.dot_general(qh[0:R0], kh[0:R0],
                                 (((1,), (1,)), ((), ())),
                                 preferred_element_type=jnp.float32)
            w0 = jnp.exp(s0)
            d0 = jnp.sum(w0, axis=1, keepdims=True)
            c0 = jnp.dot(w0.astype(bf), vh[0:R0],
                         preferred_element_type=jnp.float32)
            ctx0 = (c0 / d0).astype(bf)
            ctx1 = piece(qh[R0:R1], kh[R1:SKV], vh[R1:SKV],
                         kb0, vb0, kh[R0:R1], vh[R0:R1])
            ctx2 = piece(qh[R1:SKV], kh[R0:R1], vh[R0:R1],
                         kb0, vb0, kh[R1:SKV], vh[R1:SKV])
            ctxs.append(jnp.concatenate([ctx0, ctx1, ctx2], axis=0))
        ctx_all = jnp.concatenate(ctxs, axis=1)
        contrib = jnp.dot(ctx_all, wo_thunk(),
                          preferred_element_type=jnp.float32)
        if accumulate:
            out_ref[...] = out_ref[...] + contrib
        else:
            out_ref[...] = contrib

    def wo_slot(i):
        def thunk():
            ro[i].wait_recv()
            return commo[i]
        return thunk

    rq[0].wait_send()
    rq[1].wait_send()
    ro[0].start()
    ro[1].start()
    compute_group(my, lambda: wq_ref[...], lambda: wo_ref[...],
                  accumulate=False)

    ro[0].wait_send()
    ro[1].wait_send()
    rq[2].start()

    rq[0].wait_recv()
    compute_group(lax.rem(my + N_DEV - 1, N_DEV),
                  lambda: commq[0], wo_slot(0), accumulate=True)

    rq[2].wait_send()
    ro[2].start()

    rq[1].wait_recv()
    compute_group(lax.rem(my + 1, N_DEV),
                  lambda: commq[1], wo_slot(1), accumulate=True)

    rq[2].wait_recv()
    compute_group(lax.rem(my + 2, N_DEV),
                  lambda: commq[2], wo_slot(2), accumulate=True)

    out_ref[...] = gather_blocks(out_ref[...], _INV_ORDER)

    ro[2].wait_send()


def kernel(x, Wq, K_ext, V_ext, Wo):
    my = lax.axis_index("i")
    bf = jnp.bfloat16
    xb = x[0].astype(bf)
    wq2 = (Wq * SCALE).astype(bf)
    wo2 = Wo.astype(bf)
    kb = jnp.swapaxes(
        lax.dynamic_index_in_dim(K_ext, my, 0, keepdims=False), 0, 1).astype(bf)
    vb = jnp.swapaxes(
        lax.dynamic_index_in_dim(V_ext, my, 0, keepdims=False), 0, 1).astype(bf)

    out = pl.pallas_call(
        _body,
        out_shape=jax.ShapeDtypeStruct((SQ, D_MODEL), jnp.float32),
        in_specs=[
            pl.BlockSpec(memory_space=pltpu.VMEM),
            pl.BlockSpec(memory_space=pltpu.VMEM),
            pl.BlockSpec(memory_space=pltpu.VMEM),
            pl.BlockSpec(memory_space=pltpu.VMEM),
            pl.BlockSpec(memory_space=pltpu.VMEM),
        ],
        out_specs=pl.BlockSpec(memory_space=pltpu.VMEM),
        scratch_shapes=[
            pltpu.VMEM((N_DEV - 1, D_MODEL, D_MODEL), bf),
            pltpu.VMEM((N_DEV - 1, D_MODEL, D_MODEL), bf),
            pltpu.VMEM((R1 - R0, R1 - R0), jnp.float32),
            pltpu.VMEM((SQ, D_MODEL), bf),
            pltpu.SemaphoreType.DMA((N_DEV - 1,)),
            pltpu.SemaphoreType.DMA((N_DEV - 1,)),
            pltpu.SemaphoreType.DMA((N_DEV - 1,)),
            pltpu.SemaphoreType.DMA((N_DEV - 1,)),
        ],
        compiler_params=pltpu.CompilerParams(collective_id=0),
    )(xb, wq2, wo2, kb, vb)
    return out[None]
